# baseline (speedup 1.0000x reference)
import jax
import jax.numpy as jnp
from jax.experimental import pallas as pl
from jax.experimental.pallas import tpu as pltpu

_BEAM = 32
_SEQ = 200
_VOCAB = 1000000
_CHUNK = 512
_BLOCK_V = 98304
_CHUNKS_PER_BLOCK = _BLOCK_V // _CHUNK
_NBLK = pl.cdiv(_VOCAB, _BLOCK_V)
_NEG_INF = float("-inf")
_BIG = 2**30


def _chunk_max_kernel(dec_ref, m_ref):
    i = pl.program_id(0)
    x = dec_ref[...].reshape(_BEAM, _BLOCK_V)
    m = jnp.max(x.reshape(_BEAM, _CHUNKS_PER_BLOCK, _CHUNK), axis=2)
    m_ref[...] = m.reshape(1, _BEAM, _CHUNKS_PER_BLOCK)

    @pl.when(i == _NBLK - 1)
    def _tail():
        col = jax.lax.broadcasted_iota(jnp.int32, x.shape, 1) + i * _BLOCK_V
        xm = jnp.where(col < _VOCAB, x, 0.0)
        mm = jnp.max(xm.reshape(_BEAM, _CHUNKS_PER_BLOCK, _CHUNK), axis=2)
        m_ref[...] = mm.reshape(1, _BEAM, _CHUNKS_PER_BLOCK)


def _select_chunks_kernel(m_ref, lv_ref, scores_ref, selb_ref, selc_ref):
    m = m_ref[...]
    v = lv_ref[...] + scores_ref[...][None, :, :]
    blk = jax.lax.broadcasted_iota(jnp.int32, v.shape, 0)
    beam = jax.lax.broadcasted_iota(jnp.int32, v.shape, 1)
    loc = jax.lax.broadcasted_iota(jnp.int32, v.shape, 2)
    chunk = blk * _CHUNKS_PER_BLOCK + loc
    row = jax.lax.broadcasted_iota(jnp.int32, (_BEAM, 128), 0)

    def body(j, carry):
        v, selb, selc = carry
        top = jnp.max(v)
        hit = v == top
        b = jnp.min(jnp.where(hit, beam, _BIG))
        hit = hit & (beam == b)
        pm = jnp.max(jnp.where(hit, m, _NEG_INF))
        hit = hit & (m == pm)
        c = jnp.min(jnp.where(hit, chunk, _BIG))
        selb = jnp.where(row == j, b, selb)
        selc = jnp.where(row == j, c, selc)
        v = jnp.where((beam == b) & (chunk == c), _NEG_INF, v)
        return v, selb, selc

    zero = jnp.zeros((_BEAM, 128), jnp.int32)
    _, selb, selc = jax.lax.fori_loop(0, _BEAM, body, (v, zero, zero))
    selb_ref[...] = selb
    selc_ref[...] = selc


def _gather_kernel(bsel_ref, csel_ref, dec_ref, out_ref):
    del bsel_ref, csel_ref
    out_ref[...] = dec_ref[...]


def _merge_kernel(g_ref, lg_ref, bsel_ref, csel_ref, scores_ref, gen_ref,
                  step_ref, gen_out_ref, scores_out_ref):
    g = g_ref[...]
    colg = jax.lax.broadcasted_iota(jnp.int32, g.shape, 1)
    beammat = bsel_ref[...]
    cstart = csel_ref[...] * _CHUNK
    c32i = jax.lax.broadcasted_iota(jnp.int32, (_BEAM, _BEAM), 1)
    r32 = jax.lax.broadcasted_iota(jnp.int32, (_BEAM, _BEAM), 0)
    onehot = (beammat == c32i).astype(jnp.float32)
    chunk_scores = jax.lax.dot(onehot, scores_ref[...],
                               precision=jax.lax.Precision.HIGHEST,
                               preferred_element_type=jnp.float32)
    vocab_pos = cstart + colg
    beam_pos = beammat + jnp.zeros_like(colg)
    val = jnp.where(vocab_pos < _VOCAB, lg_ref[...] + chunk_scores, _NEG_INF)

    row200 = jax.lax.broadcasted_iota(jnp.int32, (_BEAM, 128), 0)
    R = jnp.zeros((_BEAM, _BEAM), jnp.float32)
    toks = jnp.zeros((_BEAM, 128), jnp.int32)
    sc_out = jnp.zeros((_BEAM, 128), jnp.float32)

    def extract(j, carry):
        val, R, toks, sc_out = carry
        top = jnp.max(val)
        hit = val == top
        b = jnp.min(jnp.where(hit, beam_pos, _BIG))
        hit = hit & (beam_pos == b)
        pm = jnp.max(jnp.where(hit, g, _NEG_INF))
        hit = hit & (g == pm)
        token = jnp.min(jnp.where(hit, vocab_pos, _BIG))
        R = jnp.where((r32 == j) & (c32i == b), 1.0, R)
        toks = jnp.where(row200 == j, token, toks)
        sc_out = jnp.where(row200 == j, top, sc_out)
        val = jnp.where((beam_pos == b) & (vocab_pos == token),
                        _NEG_INF, val)
        return val, R, toks, sc_out

    _, R, toks, sc_out = jax.lax.fori_loop(0, _BEAM, extract,
                                           (val, R, toks, sc_out))

    gen = gen_ref[...]
    reordered = jax.lax.dot(R, gen.astype(jnp.float32),
                            precision=jax.lax.Precision.HIGHEST,
                            preferred_element_type=jnp.float32)
    reordered = reordered.astype(jnp.int32)
    cols = jax.lax.broadcasted_iota(jnp.int32, (_BEAM, _SEQ), 1)
    step = step_ref[0, 0]
    out = jnp.where(cols < step, reordered, gen)
    out = jnp.where(cols == step, toks[:, 0:1], out)
    gen_out_ref[...] = out
    scores_out_ref[...] = sc_out


def kernel(dec_output, scores, gen_seq, step):
    chunk_max = pl.pallas_call(
        _chunk_max_kernel,
        grid=(_NBLK,),
        in_specs=[pl.BlockSpec((_BEAM, 1, _BLOCK_V), lambda i: (0, 0, i))],
        out_specs=pl.BlockSpec((1, _BEAM, _CHUNKS_PER_BLOCK),
                               lambda i: (i, 0, 0)),
        out_shape=jax.ShapeDtypeStruct((_NBLK, _BEAM, _CHUNKS_PER_BLOCK),
                                       jnp.float32),
    )(dec_output)

    scores31 = scores.reshape(_BEAM, 1)
    selb, selc = pl.pallas_call(
        _select_chunks_kernel,
        out_shape=(jax.ShapeDtypeStruct((_BEAM, 128), jnp.int32),
                   jax.ShapeDtypeStruct((_BEAM, 128), jnp.int32)),
    )(chunk_max, jnp.log(chunk_max), scores31)
    bsel = selb[:, 0]
    csel = selc[:, 0]

    gathered = pl.pallas_call(
        _gather_kernel,
        grid_spec=pltpu.PrefetchScalarGridSpec(
            num_scalar_prefetch=2,
            grid=(_BEAM,),
            in_specs=[pl.BlockSpec((1, 1, _CHUNK),
                                   lambda i, b, c: (b[i], 0, c[i]))],
            out_specs=pl.BlockSpec((1, 1, _CHUNK), lambda i, b, c: (i, 0, 0)),
        ),
        out_shape=jax.ShapeDtypeStruct((_BEAM, 1, _CHUNK), jnp.float32),
    )(bsel, csel, dec_output)
    g = gathered.reshape(_BEAM, _CHUNK)

    step_arr = jnp.asarray(step, jnp.int32).reshape(1, 1)
    gen_out, scores_mat = pl.pallas_call(
        _merge_kernel,
        out_shape=(jax.ShapeDtypeStruct((_BEAM, _SEQ), jnp.int32),
                   jax.ShapeDtypeStruct((_BEAM, 128), jnp.float32)),
    )(g, jnp.log(g), bsel.reshape(_BEAM, 1), csel.reshape(_BEAM, 1),
      scores31, gen_seq, step_arr)
    return gen_out, scores_mat[:, 0]


# speedup vs baseline: 1.0441x; 1.0441x over previous
import jax
import jax.numpy as jnp
from jax.experimental import pallas as pl
from jax.experimental.pallas import tpu as pltpu

_BEAM = 32
_SEQ = 200
_VOCAB = 1000000
_CHUNK = 512
_BLOCK_V = 65536
_CHUNKS_PER_BLOCK = _BLOCK_V // _CHUNK
_NBLK = pl.cdiv(_VOCAB, _BLOCK_V)
_NEG_INF = float("-inf")
_BIG = 2**30


def _chunk_max_kernel(dec_ref, m_ref):
    i = pl.program_id(0)
    x = dec_ref[...].reshape(_BEAM, _BLOCK_V)
    m = jnp.max(x.reshape(_BEAM, _CHUNKS_PER_BLOCK, _CHUNK), axis=2)
    m_ref[...] = m.reshape(1, _BEAM, _CHUNKS_PER_BLOCK)

    @pl.when(i == _NBLK - 1)
    def _tail():
        col = jax.lax.broadcasted_iota(jnp.int32, x.shape, 1) + i * _BLOCK_V
        xm = jnp.where(col < _VOCAB, x, 0.0)
        mm = jnp.max(xm.reshape(_BEAM, _CHUNKS_PER_BLOCK, _CHUNK), axis=2)
        m_ref[...] = mm.reshape(1, _BEAM, _CHUNKS_PER_BLOCK)


def _select_chunks_kernel(m_ref, lv_ref, scores_ref, selb_ref, selc_ref):
    m = m_ref[...]
    v = lv_ref[...] + scores_ref[...][None, :, :]
    blk = jax.lax.broadcasted_iota(jnp.int32, v.shape, 0)
    beam = jax.lax.broadcasted_iota(jnp.int32, v.shape, 1)
    loc = jax.lax.broadcasted_iota(jnp.int32, v.shape, 2)
    chunk = blk * _CHUNKS_PER_BLOCK + loc
    row = jax.lax.broadcasted_iota(jnp.int32, (_BEAM, 128), 0)

    def body(j, carry):
        v, selb, selc = carry
        top = jnp.max(v)
        hit = v == top
        b = jnp.min(jnp.where(hit, beam, _BIG))
        hit = hit & (beam == b)
        pm = jnp.max(jnp.where(hit, m, _NEG_INF))
        hit = hit & (m == pm)
        c = jnp.min(jnp.where(hit, chunk, _BIG))
        selb = jnp.where(row == j, b, selb)
        selc = jnp.where(row == j, c, selc)
        v = jnp.where((beam == b) & (chunk == c), _NEG_INF, v)
        return v, selb, selc

    zero = jnp.zeros((_BEAM, 128), jnp.int32)
    _, selb, selc = jax.lax.fori_loop(0, _BEAM, body, (v, zero, zero))
    selb_ref[...] = selb
    selc_ref[...] = selc


def _gather_kernel(bsel_ref, csel_ref, dec_ref, out_ref):
    del bsel_ref, csel_ref
    out_ref[...] = dec_ref[...]


def _merge_kernel(g_ref, lg_ref, bsel_ref, csel_ref, scores_ref, gen_ref,
                  step_ref, gen_out_ref, scores_out_ref):
    g = g_ref[...]
    colg = jax.lax.broadcasted_iota(jnp.int32, g.shape, 1)
    beammat = bsel_ref[...]
    cstart = csel_ref[...] * _CHUNK
    c32i = jax.lax.broadcasted_iota(jnp.int32, (_BEAM, _BEAM), 1)
    r32 = jax.lax.broadcasted_iota(jnp.int32, (_BEAM, _BEAM), 0)
    onehot = (beammat == c32i).astype(jnp.float32)
    chunk_scores = jax.lax.dot(onehot, scores_ref[...],
                               precision=jax.lax.Precision.HIGHEST,
                               preferred_element_type=jnp.float32)
    vocab_pos = cstart + colg
    beam_pos = beammat + jnp.zeros_like(colg)
    val = jnp.where(vocab_pos < _VOCAB, lg_ref[...] + chunk_scores, _NEG_INF)

    row200 = jax.lax.broadcasted_iota(jnp.int32, (_BEAM, 128), 0)
    R = jnp.zeros((_BEAM, _BEAM), jnp.float32)
    toks = jnp.zeros((_BEAM, 128), jnp.int32)
    sc_out = jnp.zeros((_BEAM, 128), jnp.float32)

    def extract(j, carry):
        val, R, toks, sc_out = carry
        top = jnp.max(val)
        hit = val == top
        b = jnp.min(jnp.where(hit, beam_pos, _BIG))
        hit = hit & (beam_pos == b)
        pm = jnp.max(jnp.where(hit, g, _NEG_INF))
        hit = hit & (g == pm)
        token = jnp.min(jnp.where(hit, vocab_pos, _BIG))
        R = jnp.where((r32 == j) & (c32i == b), 1.0, R)
        toks = jnp.where(row200 == j, token, toks)
        sc_out = jnp.where(row200 == j, top, sc_out)
        val = jnp.where((beam_pos == b) & (vocab_pos == token),
                        _NEG_INF, val)
        return val, R, toks, sc_out

    _, R, toks, sc_out = jax.lax.fori_loop(0, _BEAM, extract,
                                           (val, R, toks, sc_out))

    gen = gen_ref[...]
    reordered = jax.lax.dot(R, gen.astype(jnp.float32),
                            precision=jax.lax.Precision.HIGHEST,
                            preferred_element_type=jnp.float32)
    reordered = reordered.astype(jnp.int32)
    cols = jax.lax.broadcasted_iota(jnp.int32, (_BEAM, _SEQ), 1)
    step = step_ref[0, 0]
    out = jnp.where(cols < step, reordered, gen)
    out = jnp.where(cols == step, toks[:, 0:1], out)
    gen_out_ref[...] = out
    scores_out_ref[...] = sc_out


def kernel(dec_output, scores, gen_seq, step):
    chunk_max = pl.pallas_call(
        _chunk_max_kernel,
        grid=(_NBLK,),
        in_specs=[pl.BlockSpec((_BEAM, 1, _BLOCK_V), lambda i: (0, 0, i))],
        out_specs=pl.BlockSpec((1, _BEAM, _CHUNKS_PER_BLOCK),
                               lambda i: (i, 0, 0)),
        out_shape=jax.ShapeDtypeStruct((_NBLK, _BEAM, _CHUNKS_PER_BLOCK),
                                       jnp.float32),
    )(dec_output)

    scores31 = scores.reshape(_BEAM, 1)
    selb, selc = pl.pallas_call(
        _select_chunks_kernel,
        out_shape=(jax.ShapeDtypeStruct((_BEAM, 128), jnp.int32),
                   jax.ShapeDtypeStruct((_BEAM, 128), jnp.int32)),
    )(chunk_max, jnp.log(chunk_max), scores31)
    bsel = selb[:, 0]
    csel = selc[:, 0]

    gathered = pl.pallas_call(
        _gather_kernel,
        grid_spec=pltpu.PrefetchScalarGridSpec(
            num_scalar_prefetch=2,
            grid=(_BEAM,),
            in_specs=[pl.BlockSpec((1, 1, _CHUNK),
                                   lambda i, b, c: (b[i], 0, c[i]))],
            out_specs=pl.BlockSpec((1, 1, _CHUNK), lambda i, b, c: (i, 0, 0)),
        ),
        out_shape=jax.ShapeDtypeStruct((_BEAM, 1, _CHUNK), jnp.float32),
    )(bsel, csel, dec_output)
    g = gathered.reshape(_BEAM, _CHUNK)

    step_arr = jnp.asarray(step, jnp.int32).reshape(1, 1)
    gen_out, scores_mat = pl.pallas_call(
        _merge_kernel,
        out_shape=(jax.ShapeDtypeStruct((_BEAM, _SEQ), jnp.int32),
                   jax.ShapeDtypeStruct((_BEAM, 128), jnp.float32)),
    )(g, jnp.log(g), bsel.reshape(_BEAM, 1), csel.reshape(_BEAM, 1),
      scores31, gen_seq, step_arr)
    return gen_out, scores_mat[:, 0]
